# Initial kernel scaffold; baseline (speedup 1.0000x reference)
import jax
import jax.numpy as jnp
from jax.experimental import pallas as pl


def kernel(feat, edge_index, W, b):
    raise NotImplementedError("write your pallas kernel here")


# speedup vs baseline: 3.3415x; 3.3415x over previous
import functools

import jax
import jax.numpy as jnp
from jax import lax
from jax.experimental import pallas as pl
from jax.experimental.pallas import tpu as pltpu
from jax.experimental.pallas import tpu_sc as plsc

N_NODES = 10000
N_EDGES = 320000
DIM = 128
N_HOPS = 2

NC = 2
NS = 16
NW = NC * NS

CHUNK = 128
CPW = 80
PAD_E = NW * CPW * CHUNK
RPT = 632
ACC_ROWS = NS * RPT
DUMMY_ROW = N_NODES
N_BLOCKS = ACC_ROWS // 128

_sc_mesh = plsc.VectorSubcoreMesh(core_axis_name="c", subcore_axis_name="s")


def _deg_body(sidx_hbm, zeros_hbm, ones_hbm, out_hbm, acc, sidx_v, ones_v):
  c = lax.axis_index("c")
  s = lax.axis_index("s")
  w = c * NS + s
  pltpu.sync_copy(zeros_hbm, acc.at[pl.ds(s * RPT, RPT)])
  pltpu.sync_copy(ones_hbm, ones_v)
  pltpu.sync_copy(sidx_hbm.at[pl.ds(w * CPW, CPW)], sidx_v)
  plsc.subcore_barrier()

  def chunk(j, carry):
    pltpu.sync_copy(ones_v, acc.at[sidx_v.at[j]], add=True)
    return carry

  lax.fori_loop(0, CPW, chunk, 0)
  plsc.subcore_barrier()
  pltpu.sync_copy(
      acc.at[pl.ds(s * RPT, RPT)],
      out_hbm.at[pl.ds(c * ACC_ROWS + s * RPT, RPT)],
  )


_deg_call = pl.kernel(
    _deg_body,
    out_type=jax.ShapeDtypeStruct((NC * ACC_ROWS, DIM), jnp.float32),
    mesh=_sc_mesh,
    scratch_types=[
        pltpu.VMEM_SHARED((ACC_ROWS, DIM), jnp.float32),
        pltpu.VMEM((CPW, CHUNK), jnp.int32),
        pltpu.VMEM((CHUNK, DIM), jnp.float32),
    ],
)


def _hop_body(g_hbm, gidx_hbm, sidx_hbm, zeros_hbm, out_hbm,
              acc, gidx_v, sidx_v, rows_v, sem):
  c = lax.axis_index("c")
  s = lax.axis_index("s")
  w = c * NS + s
  pltpu.sync_copy(zeros_hbm, acc.at[pl.ds(s * RPT, RPT)])
  pltpu.sync_copy(gidx_hbm.at[pl.ds(w * CPW, CPW)], gidx_v)
  pltpu.sync_copy(sidx_hbm.at[pl.ds(w * CPW, CPW)], sidx_v)
  plsc.subcore_barrier()

  def chunk(j, carry):
    pltpu.async_copy(g_hbm.at[gidx_v.at[j]], rows_v, sem).wait()
    pltpu.sync_copy(rows_v, acc.at[sidx_v.at[j]], add=True)
    return carry

  lax.fori_loop(0, CPW, chunk, 0)
  plsc.subcore_barrier()
  pltpu.sync_copy(
      acc.at[pl.ds(s * RPT, RPT)],
      out_hbm.at[pl.ds(c * ACC_ROWS + s * RPT, RPT)],
  )


_hop_call = pl.kernel(
    _hop_body,
    out_type=jax.ShapeDtypeStruct((NC * ACC_ROWS, DIM), jnp.float32),
    mesh=_sc_mesh,
    scratch_types=[
        pltpu.VMEM_SHARED((ACC_ROWS, DIM), jnp.float32),
        pltpu.VMEM((CPW, CHUNK), jnp.int32),
        pltpu.VMEM((CPW, CHUNK), jnp.int32),
        pltpu.VMEM((CHUNK, DIM), jnp.float32),
        pltpu.SemaphoreType.DMA,
    ],
)


def _norm_body(degp_ref, feat_ref, norm_ref, norm2_ref, g0_ref):
  deg = degp_ref[0, :, :1] + degp_ref[1, :, :1] + 1.0
  norm = lax.rsqrt(deg)
  norm_ref[...] = norm
  norm2_ref[...] = 1.0 / deg
  g0_ref[...] = feat_ref[...] * norm


def _norm_call(degp, featp):
  return pl.pallas_call(
      _norm_body,
      grid=(N_BLOCKS,),
      in_specs=[
          pl.BlockSpec((NC, 128, DIM), lambda i: (0, i, 0)),
          pl.BlockSpec((128, DIM), lambda i: (i, 0)),
      ],
      out_specs=[
          pl.BlockSpec((128, 1), lambda i: (i, 0)),
          pl.BlockSpec((128, 1), lambda i: (i, 0)),
          pl.BlockSpec((128, DIM), lambda i: (i, 0)),
      ],
      out_shape=[
          jax.ShapeDtypeStruct((ACC_ROWS, 1), jnp.float32),
          jax.ShapeDtypeStruct((ACC_ROWS, 1), jnp.float32),
          jax.ShapeDtypeStruct((ACC_ROWS, DIM), jnp.float32),
      ],
  )(degp, featp)


def _comb_body(part_ref, g_ref, norm_ref, norm2_ref, h_ref, gn_ref):
  t = part_ref[0] + part_ref[1] + g_ref[...]
  h_ref[...] = t * norm_ref[...]
  gn_ref[...] = t * norm2_ref[...]


def _comb_call(part, g, norm, norm2):
  return pl.pallas_call(
      _comb_body,
      grid=(N_BLOCKS,),
      in_specs=[
          pl.BlockSpec((NC, 128, DIM), lambda i: (0, i, 0)),
          pl.BlockSpec((128, DIM), lambda i: (i, 0)),
          pl.BlockSpec((128, 1), lambda i: (i, 0)),
          pl.BlockSpec((128, 1), lambda i: (i, 0)),
      ],
      out_specs=[
          pl.BlockSpec((128, DIM), lambda i: (i, 0)),
          pl.BlockSpec((128, DIM), lambda i: (i, 0)),
      ],
      out_shape=[
          jax.ShapeDtypeStruct((ACC_ROWS, DIM), jnp.float32),
          jax.ShapeDtypeStruct((ACC_ROWS, DIM), jnp.float32),
      ],
  )(part, g, norm, norm2)


def _mm_body(x_ref, wt_ref, b_ref, out_ref):
  out_ref[...] = (
      jnp.dot(x_ref[...], wt_ref[...], preferred_element_type=jnp.float32)
      + b_ref[...]
  )


def _mm_call(x, wt, b2):
  k = x.shape[1]
  return pl.pallas_call(
      _mm_body,
      grid=(N_BLOCKS,),
      in_specs=[
          pl.BlockSpec((128, k), lambda i: (i, 0)),
          pl.BlockSpec((k, DIM), lambda i: (0, 0)),
          pl.BlockSpec((1, DIM), lambda i: (0, 0)),
      ],
      out_specs=pl.BlockSpec((128, DIM), lambda i: (i, 0)),
      out_shape=jax.ShapeDtypeStruct((ACC_ROWS, DIM), jnp.float32),
  )(x, wt, b2)


@jax.jit
def kernel(feat, edge_index, W, b):
  src = edge_index[0]
  dst = edge_index[1]
  n_pad = PAD_E - N_EDGES
  pad_gather = jnp.zeros((n_pad,), dtype=jnp.int32)
  pad_scatter = jnp.full((n_pad,), DUMMY_ROW, dtype=jnp.int32)

  gidx_a = jnp.concatenate([dst, pad_gather]).reshape(NW * CPW, CHUNK)
  sidx_a = jnp.concatenate([src, pad_scatter]).reshape(NW * CPW, CHUNK)
  gidx_b = jnp.concatenate([src, pad_gather]).reshape(NW * CPW, CHUNK)
  sidx_b = jnp.concatenate([dst, pad_scatter]).reshape(NW * CPW, CHUNK)

  featp = jnp.pad(feat, ((0, ACC_ROWS - N_NODES), (0, 0)))
  zeros128 = jnp.zeros((RPT, DIM), jnp.float32)
  ones128 = jnp.ones((CHUNK, DIM), jnp.float32)

  degp = _deg_call(sidx_b, zeros128, ones128).reshape(NC, ACC_ROWS, DIM)
  norm, norm2, g0 = _norm_call(degp, featp)

  fstack = [featp]
  g = g0
  for hop in range(2 * N_HOPS):
    gidx, sidx = (gidx_a, sidx_a) if hop < N_HOPS else (gidx_b, sidx_b)
    part = _hop_call(g, gidx, sidx, zeros128).reshape(NC, ACC_ROWS, DIM)
    h, g = _comb_call(part, g, norm, norm2)
    fstack.append(h)

  x = jnp.concatenate(fstack, axis=1)
  out = _mm_call(x, W.T, b.reshape(1, DIM))
  return out[:N_NODES]
